# baseline (speedup 1.0000x reference)
import functools

import jax
import jax.numpy as jnp
from jax import lax
from jax.experimental import pallas as pl
from jax.experimental.pallas import tpu as pltpu
from jax.experimental.pallas import tpu_sc as plsc

N_CLS = 1000
PRE = 5
NCTX = 16
TOT = 77
SUF = TOT - PRE - NCTX
D = 512
NW = 32
LEN = 32
NSLOT = 3

_mesh = plsc.VectorSubcoreMesh(core_axis_name="c", subcore_axis_name="s")


@functools.partial(
    pl.kernel,
    mesh=_mesh,
    out_type=jax.ShapeDtypeStruct((TOT, N_CLS, D), jnp.float32),
    scratch_types=(
        [pltpu.VMEM((LEN,), jnp.int32)] * NSLOT
        + [pltpu.VMEM((LEN, D), jnp.float32)] * NSLOT
        + [pltpu.SemaphoreType.DMA] * (2 * NSLOT)
    ),
)
def _assemble(ctx_hbm, pre_hbm, suf_hbm, out_hbm,
              i0, i1, i2, r0, r1, r2, g0, g1, g2, o0, o1, o2):
    idx = [i0, i1, i2]
    rows = [r0, r1, r2]
    gsem = [g0, g1, g2]
    osem = [o0, o1, o2]
    wid = lax.axis_index("s") * 2 + lax.axis_index("c")
    cstart = jnp.minimum(wid * LEN, N_CLS - LEN)
    j16 = lax.iota(jnp.int32, 16)
    lo = cstart + j16
    hi = cstart + 16 + j16

    def table(p):
        return pre_hbm if p < PRE else (ctx_hbm if p < PRE + NCTX else suf_hbm)

    def build_idx(p, s):
        if p < PRE:
            a, b = p * N_CLS + lo, p * N_CLS + hi
        elif p < PRE + NCTX:
            a = b = (p - PRE) + j16 * 0
        else:
            t = p - PRE - NCTX
            a, b = lo * SUF + t, hi * SUF + t
        idx[s][pl.ds(0, 16)] = a
        idx[s][pl.ds(16, 16)] = b

    def gather_copy(p, s):
        return pltpu.make_async_copy(table(p).at[idx[s]], rows[s], gsem[s])

    def out_copy(p, s):
        return pltpu.make_async_copy(
            rows[s], out_hbm.at[p, pl.ds(cstart, LEN)], osem[s])

    build_idx(0, 0)
    gather_copy(0, 0).start()
    build_idx(1, 1)
    gather_copy(1, 1).start()
    for p in range(TOT):
        s = p % NSLOT
        gather_copy(p, s).wait()
        out_copy(p, s).start()
        if p + 2 < TOT:
            s2 = (p + 2) % NSLOT
            if p - 1 >= 0:
                out_copy(p - 1, s2).wait()
            build_idx(p + 2, s2)
            gather_copy(p + 2, s2).start()
    for p in range(TOT - NSLOT, TOT):
        out_copy(p, p % NSLOT).wait()


def kernel(ctx, token_prefix, token_suffix):
    pre2d = jnp.transpose(token_prefix, (1, 0, 2)).reshape(PRE * N_CLS, D)
    suf2d = token_suffix.reshape(N_CLS * SUF, D)
    out_t = _assemble(ctx, pre2d, suf2d)
    return jnp.transpose(out_t, (1, 0, 2))


# speedup vs baseline: 2.8362x; 2.8362x over previous
import functools

import jax
import jax.numpy as jnp
from jax import lax
from jax.experimental import pallas as pl
from jax.experimental.pallas import tpu as pltpu
from jax.experimental.pallas import tpu_sc as plsc

N_CLS = 1000
PRE = 5
NCTX = 16
TOT = 77
SUF = TOT - PRE - NCTX
D = 512
NW = 32
CHUNK = 112
NCHUNK = 9
TAIL = N_CLS - 8 * CHUNK

PRE_ROWS = PRE * N_CLS
CTX_ROW0 = PRE_ROWS
SUF_ROW0 = (PRE + NCTX) * N_CLS

_mesh = plsc.VectorSubcoreMesh(core_axis_name="c", subcore_axis_name="s")


@functools.partial(
    pl.kernel,
    mesh=_mesh,
    out_type=jax.ShapeDtypeStruct((TOT * N_CLS, D), jnp.float32),
    scratch_types=[
        pltpu.VMEM((CHUNK,), jnp.int32),
        pltpu.VMEM((CHUNK,), jnp.int32),
        pltpu.VMEM((16,), jnp.int32),
        pltpu.VMEM((CHUNK, D), jnp.float32),
        pltpu.VMEM((CHUNK, D), jnp.float32),
        pltpu.VMEM((16, D), jnp.float32),
        pltpu.SemaphoreType.DMA,
        pltpu.SemaphoreType.DMA,
        pltpu.SemaphoreType.DMA,
        pltpu.SemaphoreType.DMA,
        pltpu.SemaphoreType.DMA,
    ],
)
def _assemble(ctx_hbm, pre_hbm, suf_hbm, out_hbm,
              ia, ib, ic, ra, rb, crep, ga, gb, oa, ob, oc):
    idx = [ia, ib]
    rows = [ra, rb]
    gsem = [ga, gb]
    osem = [oa, ob]
    wid = lax.axis_index("s") * 2 + lax.axis_index("c")
    j16 = lax.iota(jnp.int32, 16)

    a = jnp.minimum(wid * 160, PRE_ROWS - 160)
    pcp = [
        pltpu.make_async_copy(
            pre_hbm.at[pl.ds(a + 80 * h, 80)], rows[h].at[pl.ds(0, 80)],
            gsem[h])
        for h in range(2)
    ]
    ocp = [
        pltpu.make_async_copy(
            rows[h].at[pl.ds(0, 80)], out_hbm.at[pl.ds(a + 80 * h, 80)],
            osem[h])
        for h in range(2)
    ]
    pcp[0].start()
    pcp[1].start()
    for h in range(2):
        pcp[h].wait()
        ocp[h].start()

    k = wid // 2
    ic[pl.ds(0, 16)] = k + j16 * 0
    cg = pltpu.make_async_copy(ctx_hbm.at[ic], crep, oc)
    cg.start()
    cg.wait()
    cbase = CTX_ROW0 + k * N_CLS + jnp.minimum((wid % 2) * 504, N_CLS - 504)
    ccp = [
        pltpu.make_async_copy(
            crep, out_hbm.at[pl.ds(cbase + 16 * u, 16)], oc)
        for u in range(31)
    ] + [
        pltpu.make_async_copy(
            crep.at[pl.ds(0, 8)],
            out_hbm.at[pl.ds(cbase + 16 * 31, 8)], oc)
    ]
    for cp in ccp:
        cp.start()
    for h in range(2):
        ocp[h].wait()
    for cp in ccp:
        cp.wait()

    def build_sidx(s, ci, t):
        c0 = CHUNK * ci
        for n in range(7):
            c = jnp.minimum(c0 + 16 * n + j16, N_CLS - 1)
            idx[s][pl.ds(16 * n, 16)] = c * SUF + t

    def g_copy(s):
        return pltpu.make_async_copy(suf_hbm.at[idx[s]], rows[s], gsem[s])

    def o_copy(s, ci, t):
        c0 = CHUNK * ci
        ln = CHUNK if ci < NCHUNK - 1 else TAIL
        return pltpu.make_async_copy(
            rows[s].at[pl.ds(0, ln)],
            out_hbm.at[pl.ds(SUF_ROW0 + t * N_CLS + c0, ln)], osem[s])

    def do_plane(t):
        build_sidx(0, 0, t)
        g_copy(0).start()
        build_sidx(1, 1, t)
        g_copy(1).start()
        for ci in range(NCHUNK):
            s = ci % 2
            g_copy(s).wait()
            o_copy(s, ci, t).start()
            if ci + 2 < NCHUNK:
                o_copy(s, ci, t).wait()
                build_sidx(s, ci + 2, t)
                g_copy(s).start()
        for ci in (NCHUNK - 2, NCHUNK - 1):
            o_copy(ci % 2, ci, t).wait()

    do_plane(wid)

    @pl.when(wid < SUF - 32)
    def _():
        do_plane(wid + 32)


def kernel(ctx, token_prefix, token_suffix):
    pre2d = jnp.transpose(token_prefix, (1, 0, 2)).reshape(PRE * N_CLS, D)
    suf2d = token_suffix.reshape(N_CLS * SUF, D)
    out2d = _assemble(ctx, pre2d, suf2d)
    out_t = out2d.reshape(TOT, N_CLS, D)
    return jnp.transpose(out_t, (1, 0, 2))
